# baseline (speedup 1.0000x reference)
import jax
import jax.numpy as jnp
from jax import lax
from jax.experimental import pallas as pl
from jax.experimental.pallas import tpu as pltpu

_B, _S, _D = 1, 2048, 1024
_E = 8
_FF, _FF_SH = 256, 512
_T = _B * _S


def _silu(x):
    return x * (1.0 / (1.0 + jnp.exp(-x)))


_CH = _T // _E


def _moe_body(flat_ref, router_w_ref, wg_ref, wu_ref, wd_ref,
              shg_ref, shu_ref, shd_ref,
              out_ref, logits_ref, comb_ref, flatb_ref,
              shgb_ref, shub_ref, shdb_ref):
    e = pl.program_id(0)

    @pl.when(e == 0)
    def _prologue():
        flat = flat_ref[...]
        flatb_ref[...] = flat.astype(jnp.bfloat16)
        shgb_ref[...] = shg_ref[...].astype(jnp.bfloat16)
        shub_ref[...] = shu_ref[...].astype(jnp.bfloat16)
        shdb_ref[...] = shd_ref[...].astype(jnp.bfloat16)
        logits = jnp.dot(flat, router_w_ref[...],
                         preferred_element_type=jnp.float32)
        logits_ref[...] = logits
        lt = jnp.transpose(logits)
        lmax = jnp.max(lt, axis=0, keepdims=True)
        p = jnp.exp(lt - lmax)
        sub = lax.broadcasted_iota(jnp.int32, (_E, _T), 0)
        m1 = jnp.max(p, axis=0, keepdims=True)
        i1 = jnp.min(jnp.where(p == m1, sub, _E), axis=0, keepdims=True)
        p2 = jnp.where(sub == i1, -jnp.inf, p)
        m2 = jnp.max(p2, axis=0, keepdims=True)
        i2 = jnp.min(jnp.where(p2 == m2, sub, _E), axis=0, keepdims=True)
        s = m1 + m2
        comb_ref[...] = jnp.where(sub == i1, m1 / s, 0.0) + jnp.where(
            sub == i2, m2 / s, 0.0)

    fb = flatb_ref[...]
    col = jnp.transpose(jnp.sum(jnp.where(
        lax.broadcasted_iota(jnp.int32, (_E, _T), 0) == e, comb_ref[...], 0.0),
        axis=0, keepdims=True))
    g = jnp.dot(fb, wg_ref[0].astype(jnp.bfloat16),
                preferred_element_type=jnp.float32)
    u = jnp.dot(fb, wu_ref[0].astype(jnp.bfloat16),
                preferred_element_type=jnp.float32)
    routed = col * jnp.dot((_silu(g) * u).astype(jnp.bfloat16),
                           wd_ref[0].astype(jnp.bfloat16),
                           preferred_element_type=jnp.float32)

    fbc = flatb_ref[pl.ds(e * _CH, _CH), :]
    sg = jnp.dot(fbc, shgb_ref[...], preferred_element_type=jnp.float32)
    su = jnp.dot(fbc, shub_ref[...], preferred_element_type=jnp.float32)
    sh = jnp.dot((_silu(sg) * su).astype(jnp.bfloat16), shdb_ref[...],
                 preferred_element_type=jnp.float32)

    @pl.when(e == 0)
    def _init():
        out_ref[...] = routed

    @pl.when(e > 0)
    def _acc():
        out_ref[...] += routed

    out_ref[pl.ds(e * _CH, _CH), :] += sh


@jax.jit
def kernel(hidden_states, router_w, w_gate, w_up, w_down,
           sh_gate, sh_up, sh_down):
    flat = hidden_states.reshape(_T, _D)
    out, logits = pl.pallas_call(
        _moe_body,
        grid=(_E,),
        in_specs=[
            pl.BlockSpec((_T, _D), lambda e: (0, 0)),
            pl.BlockSpec((_D, _E), lambda e: (0, 0)),
            pl.BlockSpec((1, _D, _FF), lambda e: (e, 0, 0)),
            pl.BlockSpec((1, _D, _FF), lambda e: (e, 0, 0)),
            pl.BlockSpec((1, _FF, _D), lambda e: (e, 0, 0)),
            pl.BlockSpec((_D, _FF_SH), lambda e: (0, 0)),
            pl.BlockSpec((_D, _FF_SH), lambda e: (0, 0)),
            pl.BlockSpec((_FF_SH, _D), lambda e: (0, 0)),
        ],
        out_specs=[
            pl.BlockSpec((_T, _D), lambda e: (0, 0)),
            pl.BlockSpec((_T, _E), lambda e: (0, 0)),
        ],
        out_shape=[
            jax.ShapeDtypeStruct((_T, _D), jnp.float32),
            jax.ShapeDtypeStruct((_T, _E), jnp.float32),
        ],
        scratch_shapes=[pltpu.VMEM((_E, _T), jnp.float32),
                        pltpu.VMEM((_T, _D), jnp.bfloat16),
                        pltpu.VMEM((_D, _FF_SH), jnp.bfloat16),
                        pltpu.VMEM((_D, _FF_SH), jnp.bfloat16),
                        pltpu.VMEM((_FF_SH, _D), jnp.bfloat16)],
        compiler_params=pltpu.CompilerParams(
            dimension_semantics=("arbitrary",),
        ),
    )(flat, router_w, w_gate, w_up, w_down, sh_gate, sh_up, sh_down)
    return out.reshape(_B, _S, _D), logits
